# baseline (speedup 1.0000x reference)
import functools

import jax
import jax.numpy as jnp
from jax.experimental import pallas as pl
from jax.experimental.pallas import tpu as pltpu


def _body(cb_ref, mid_ref, x_ref, qt_ref, q_ref, o_ref, *, n_pos):
    xb = x_ref[0]
    yn = jnp.dot(xb, qt_ref[...], preferred_element_type=jnp.float32)
    a = jnp.abs(yn)
    q = jnp.full_like(a, cb_ref[0, 0])
    for j in range(1, n_pos):
        q = jnp.where(a > mid_ref[0, j - 1], cb_ref[0, j], q)
    yq = jnp.where(yn < 0.0, -q, q)
    o_ref[0] = jnp.dot(yq, q_ref[...], preferred_element_type=jnp.float32)


def kernel(x, rotation, codebook):
    b, s, dim = x.shape
    scale = 1.0 / (dim ** 0.5)

    k = codebook.shape[0]
    n_pos = k // 2
    cb_pos = codebook[n_pos:].reshape(1, n_pos)
    mids = (cb_pos[:, :-1] + cb_pos[:, 1:]) * 0.5

    qt_s = rotation.T * (1.0 / scale)
    q_s = rotation * scale

    out = pl.pallas_call(
        functools.partial(_body, n_pos=n_pos),
        grid=(b,),
        in_specs=[
            pl.BlockSpec(memory_space=pltpu.SMEM),
            pl.BlockSpec(memory_space=pltpu.SMEM),
            pl.BlockSpec((1, s, dim), lambda i: (i, 0, 0)),
            pl.BlockSpec((dim, dim), lambda i: (0, 0)),
            pl.BlockSpec((dim, dim), lambda i: (0, 0)),
        ],
        out_specs=pl.BlockSpec((1, s, dim), lambda i: (i, 0, 0)),
        out_shape=jax.ShapeDtypeStruct((b, s, dim), jnp.float32),
        compiler_params=pltpu.CompilerParams(
            dimension_semantics=("parallel",),
        ),
    )(cb_pos, mids, x, qt_s, q_s)
    return out


# speedup vs baseline: 1.0681x; 1.0681x over previous
import functools

import jax
import jax.numpy as jnp
from jax import lax
from jax.experimental import pallas as pl
from jax.experimental.pallas import tpu as pltpu


def _body(cb_ref, x_ref, rot_ref, o_ref, *, n_pos, scale):
    rot = rot_ref[...]
    xb = x_ref[0]
    y = lax.dot_general(xb, rot, (((1,), (1,)), ((), ())),
                        preferred_element_type=jnp.float32)
    a = jnp.abs(y)
    c = [cb_ref[0, n_pos + j] * scale for j in range(n_pos)]
    mids = [(c[j - 1] + c[j]) * 0.5 for j in range(1, n_pos)]
    q = jnp.full_like(a, c[0])
    for j in range(1, n_pos):
        q = jnp.where(a > mids[j - 1], c[j], q)
    yq = jnp.where(y < 0.0, -q, q)
    o_ref[0] = jnp.dot(yq, rot, preferred_element_type=jnp.float32)


def kernel(x, rotation, codebook):
    b, s, dim = x.shape
    scale = 1.0 / (dim ** 0.5)
    k = codebook.shape[0]
    n_pos = k // 2
    cb2 = codebook.reshape(1, k)

    out = pl.pallas_call(
        functools.partial(_body, n_pos=n_pos, scale=scale),
        grid=(b,),
        in_specs=[
            pl.BlockSpec(memory_space=pltpu.SMEM),
            pl.BlockSpec((1, s, dim), lambda i: (i, 0, 0)),
            pl.BlockSpec((dim, dim), lambda i: (0, 0)),
        ],
        out_specs=pl.BlockSpec((1, s, dim), lambda i: (i, 0, 0)),
        out_shape=jax.ShapeDtypeStruct((b, s, dim), jnp.float32),
        compiler_params=pltpu.CompilerParams(
            dimension_semantics=("parallel",),
        ),
    )(cb2, x, rotation)
    return out
